# Initial kernel scaffold; baseline (speedup 1.0000x reference)
import jax
import jax.numpy as jnp
from jax.experimental import pallas as pl


def kernel(x, edge_index, W, b):
    raise NotImplementedError("write your pallas kernel here")


# speedup vs baseline: 4.9342x; 4.9342x over previous
import functools

import jax
import jax.numpy as jnp
from jax import lax
from jax.experimental import pallas as pl
from jax.experimental.pallas import tpu as pltpu
from jax.experimental.pallas import tpu_sc as plsc

NC = 2
NS = 16
DEGW = 16


def _make_deg_kernel(E, N, C):
    per_tile = E // (NC * NS)
    n_chunks = per_tile // C
    rpt = N // NS
    mesh = plsc.VectorSubcoreMesh(core_axis_name="c", subcore_axis_name="s")

    @functools.partial(
        pl.kernel,
        out_type=jax.ShapeDtypeStruct((NC, N), jnp.float32),
        mesh=mesh,
        scratch_types=[
            pltpu.VMEM((C,), jnp.int32),
            pltpu.VMEM((C,), jnp.float32),
            pltpu.VMEM((rpt,), jnp.float32),
            pltpu.VMEM_SHARED((N,), jnp.float32),
        ],
    )
    def deg_kernel(dst_hbm, out_hbm, idx_v, ones_v, zeros_v, acc_sh):
        cid = lax.axis_index("c")
        sid = lax.axis_index("s")
        base = (cid * NS + sid) * per_tile
        row0 = sid * rpt

        for i in range(C // 16):
            ones_v[pl.ds(i * 16, 16)] = jnp.ones((16,), jnp.float32)

        def fill_zeros(i, _):
            zeros_v[pl.ds(i * 16, 16)] = jnp.zeros((16,), jnp.float32)
            return 0

        lax.fori_loop(0, rpt // 16, fill_zeros, 0)
        pltpu.sync_copy(zeros_v, acc_sh.at[pl.ds(row0, rpt)])
        plsc.subcore_barrier()

        def body(j, _):
            pltpu.sync_copy(dst_hbm.at[pl.ds(base + j * C, C)], idx_v)
            pltpu.sync_copy(ones_v, acc_sh.at[idx_v], add=True)
            return 0

        lax.fori_loop(0, n_chunks, body, 0)
        plsc.subcore_barrier()
        pltpu.sync_copy(acc_sh.at[pl.ds(row0, rpt)],
                        out_hbm.at[cid, pl.ds(row0, rpt)])

    return deg_kernel


def _make_prop_kernel(E, N, D, C):
    per_tile = E // (NC * NS)
    n_chunks = per_tile // C
    rpt = N // NS
    zrows = 128
    mesh = plsc.VectorSubcoreMesh(core_axis_name="c", subcore_axis_name="s")

    @functools.partial(
        pl.kernel,
        out_type=jax.ShapeDtypeStruct((NC, N, D), jnp.float32),
        mesh=mesh,
        scratch_types=[
            pltpu.VMEM((C,), jnp.int32),
            pltpu.VMEM((C,), jnp.int32),
            pltpu.VMEM((C, D), jnp.float32),
            pltpu.VMEM((zrows, D), jnp.float32),
            pltpu.VMEM_SHARED((N, D), jnp.float32),
            pltpu.SemaphoreType.DMA,
        ],
    )
    def prop_kernel(h_hbm, src_hbm, dst_hbm, out_hbm,
                    src_v, dst_v, rows_v, zeros_v, acc_sh, sem):
        cid = lax.axis_index("c")
        sid = lax.axis_index("s")
        base = (cid * NS + sid) * per_tile
        row0 = sid * rpt

        def fill_zeros(i, _):
            zeros_v[i, :] = jnp.zeros((D,), jnp.float32)
            return 0

        lax.fori_loop(0, zrows, fill_zeros, 0)

        def zcopy(i, _):
            pltpu.sync_copy(zeros_v, acc_sh.at[pl.ds(row0 + i * zrows, zrows)])
            return 0

        lax.fori_loop(0, rpt // zrows, zcopy, 0)
        plsc.subcore_barrier()

        def body(j, _):
            eb = base + j * C
            pltpu.sync_copy(src_hbm.at[pl.ds(eb, C)], src_v)
            pltpu.sync_copy(dst_hbm.at[pl.ds(eb, C)], dst_v)
            pltpu.async_copy(h_hbm.at[src_v], rows_v, sem).wait()
            pltpu.sync_copy(rows_v, acc_sh.at[dst_v], add=True)
            return 0

        lax.fori_loop(0, n_chunks, body, 0)
        plsc.subcore_barrier()
        pltpu.sync_copy(acc_sh.at[pl.ds(row0, rpt)],
                        out_hbm.at[cid, pl.ds(row0, rpt)])

    return prop_kernel


def _norm_scale(degp, x, R=1024):
    N, D = x.shape

    def body(degp_ref, x_ref, h_ref, norm_ref):
        deg = degp_ref[0] + degp_ref[1]
        norm = jnp.where(deg > 0, lax.rsqrt(jnp.maximum(deg, 1.0)), 0.0)
        h_ref[...] = x_ref[...] * norm
        norm_ref[...] = norm

    return pl.pallas_call(
        body,
        grid=(N // R,),
        in_specs=[
            pl.BlockSpec((NC, R, 1), lambda i: (0, i, 0)),
            pl.BlockSpec((R, D), lambda i: (i, 0)),
        ],
        out_specs=[
            pl.BlockSpec((R, D), lambda i: (i, 0)),
            pl.BlockSpec((R, 1), lambda i: (i, 0)),
        ],
        out_shape=[
            jax.ShapeDtypeStruct((N, D), jnp.float32),
            jax.ShapeDtypeStruct((N, 1), jnp.float32),
        ],
    )(degp, x)


def _combine_scale2(p, norm, R=1024):
    _, N, D = p.shape

    def body(p_ref, norm_ref, o_ref):
        n = norm_ref[...]
        o_ref[...] = (p_ref[0] + p_ref[1]) * (n * n)

    return pl.pallas_call(
        body,
        grid=(N // R,),
        in_specs=[
            pl.BlockSpec((NC, R, D), lambda i: (0, i, 0)),
            pl.BlockSpec((R, 1), lambda i: (i, 0)),
        ],
        out_specs=pl.BlockSpec((R, D), lambda i: (i, 0)),
        out_shape=jax.ShapeDtypeStruct((N, D), jnp.float32),
    )(p, norm)


def _combine_linear(p, norm, W, b, R=1024):
    _, N, D = p.shape
    DO = W.shape[1]

    def body(p_ref, norm_ref, w_ref, b_ref, o_ref):
        h = (p_ref[0] + p_ref[1]) * norm_ref[...]
        o_ref[...] = (
            jnp.dot(h, w_ref[...], preferred_element_type=jnp.float32)
            + b_ref[...]
        )

    return pl.pallas_call(
        body,
        grid=(N // R,),
        in_specs=[
            pl.BlockSpec((NC, R, D), lambda i: (0, i, 0)),
            pl.BlockSpec((R, 1), lambda i: (i, 0)),
            pl.BlockSpec((D, DO), lambda i: (0, 0)),
            pl.BlockSpec((1, DO), lambda i: (0, 0)),
        ],
        out_specs=pl.BlockSpec((R, DO), lambda i: (i, 0)),
        out_shape=jax.ShapeDtypeStruct((N, DO), jnp.float32),
    )(p, norm, W, b.reshape(1, DO))


def kernel(x, edge_index, W, b):
    N, D = x.shape
    E = edge_index.shape[1]
    NP = ((N + NS * 128 - 1) // (NS * 128)) * (NS * 128)
    assert E % (NC * NS) == 0
    C = 80
    assert (E // (NC * NS)) % C == 0 and (NP // NS) % 128 == 0

    src = edge_index[0]
    dst = edge_index[1]
    xp = jnp.pad(x, ((0, NP - N), (0, 0))) if NP != N else x

    degp = _make_deg_kernel(E, NP, C)(dst).reshape(NC, NP, 1)
    h1, norm = _norm_scale(degp, xp)
    prop = _make_prop_kernel(E, NP, D, C)
    p1 = prop(h1, src, dst)
    h2 = _combine_scale2(p1, norm)
    p2 = prop(h2, src, dst)
    out = _combine_linear(p2, norm, W, b)
    return out[:N] if NP != N else out


# speedup vs baseline: 12.2879x; 2.4903x over previous
import functools

import jax
import jax.numpy as jnp
from jax import lax
from jax.experimental import pallas as pl
from jax.experimental.pallas import tpu as pltpu
from jax.experimental.pallas import tpu_sc as plsc

NC = 2
NS = 16
NW = NC * NS


def _make_deg_kernel(N, NCH, C):
    rpt = N // NS
    mesh = plsc.VectorSubcoreMesh(core_axis_name="c", subcore_axis_name="s")

    @functools.partial(
        pl.kernel,
        out_type=jax.ShapeDtypeStruct((NC, N), jnp.float32),
        mesh=mesh,
        scratch_types=[
            pltpu.VMEM((NCH, C), jnp.int32),
            pltpu.VMEM((C,), jnp.int32),
            pltpu.VMEM((C,), jnp.float32),
            pltpu.VMEM((rpt,), jnp.float32),
            pltpu.VMEM_SHARED((N,), jnp.float32),
        ],
    )
    def deg_kernel(dst_hbm, out_hbm, dsts_v, dbuf, ones_v, zeros_v, acc_sh):
        cid = lax.axis_index("c")
        sid = lax.axis_index("s")
        wid = cid * NS + sid
        row0 = sid * rpt

        pltpu.sync_copy(dst_hbm.at[wid], dsts_v)

        for i in range(C // 16):
            ones_v[pl.ds(i * 16, 16)] = jnp.ones((16,), jnp.float32)

        def fill_zeros(i, _):
            zeros_v[pl.ds(i * 16, 16)] = jnp.zeros((16,), jnp.float32)
            return 0

        lax.fori_loop(0, rpt // 16, fill_zeros, 0)
        pltpu.sync_copy(zeros_v, acc_sh.at[pl.ds(row0, rpt)])
        plsc.subcore_barrier()

        def body(c, _):
            for k in range(C // 16):
                dbuf[pl.ds(16 * k, 16)] = dsts_v[c, pl.ds(16 * k, 16)]
            pltpu.sync_copy(ones_v, acc_sh.at[dbuf], add=True)
            return 0

        lax.fori_loop(0, NCH, body, 0)
        plsc.subcore_barrier()
        pltpu.sync_copy(acc_sh.at[pl.ds(row0, rpt)],
                        out_hbm.at[cid, pl.ds(row0, rpt)])

    return deg_kernel


def _make_prop_kernel(N, D, NCH, C):
    rpt = N // NS
    zrows = 16
    mesh = plsc.VectorSubcoreMesh(core_axis_name="c", subcore_axis_name="s")
    assert NCH % 4 == 0 and NCH >= 8

    @functools.partial(
        pl.kernel,
        out_type=jax.ShapeDtypeStruct((NC, N, D), jnp.float32),
        mesh=mesh,
        scratch_types=(
            [pltpu.VMEM((C,), jnp.int32) for _ in range(4)]
            + [pltpu.VMEM((C,), jnp.int32) for _ in range(4)]
            + [pltpu.VMEM((C, D), jnp.float32) for _ in range(2)]
            + [pltpu.VMEM((zrows, D), jnp.float32),
               pltpu.VMEM_SHARED((N, D), jnp.float32)]
            + [pltpu.SemaphoreType.DMA for _ in range(6)]
        ),
    )
    def prop_kernel(h_hbm, src_hbm, dst_hbm, out_hbm,
                    is0, is1, is2, is3, id0, id1, id2, id3,
                    rows0, rows1, zeros_v, acc_sh,
                    semi0, semi1, semi2, semi3, semg0, semg1):
        isrc = (is0, is1, is2, is3)
        idst = (id0, id1, id2, id3)
        semi = (semi0, semi1, semi2, semi3)
        rows = (rows0, rows1)
        semg = (semg0, semg1)
        cid = lax.axis_index("c")
        sid = lax.axis_index("s")
        base = (cid * NS + sid) * (NCH * C)
        row0 = sid * rpt

        def start_idx(c, slot):
            pltpu.async_copy(src_hbm.at[pl.ds(base + c * C, C)],
                             isrc[slot], semi[slot])
            pltpu.async_copy(dst_hbm.at[pl.ds(base + c * C, C)],
                             idst[slot], semi[slot])

        def wait_idx(c, slot):
            pltpu.make_async_copy(src_hbm.at[pl.ds(base + c * C, C)],
                                  isrc[slot], semi[slot]).wait()
            pltpu.make_async_copy(dst_hbm.at[pl.ds(base + c * C, C)],
                                  idst[slot], semi[slot]).wait()

        for q in range(4):
            start_idx(q, q)

        def fill_zeros(i, _):
            zeros_v[i, :] = jnp.zeros((D,), jnp.float32)
            return 0

        lax.fori_loop(0, zrows, fill_zeros, 0)

        def zcopy(i, _):
            pltpu.sync_copy(zeros_v, acc_sh.at[pl.ds(row0 + i * zrows, zrows)])
            return 0

        lax.fori_loop(0, rpt // zrows, zcopy, 0)

        for q in range(2):
            wait_idx(q, q)
            pltpu.async_copy(h_hbm.at[isrc[q]], rows[q], semg[q])
        plsc.subcore_barrier()

        def quad(j, _):
            for q in range(4):
                c = 4 * j + q
                rp = q % 2
                pltpu.make_async_copy(h_hbm.at[isrc[q]], rows[rp],
                                      semg[rp]).wait()
                pltpu.sync_copy(rows[rp], acc_sh.at[idst[q]], add=True)

                @pl.when(c + 4 < NCH)
                def _():
                    start_idx(c + 4, q)

                @pl.when(c + 2 < NCH)
                def _():
                    iq = (q + 2) % 4
                    wait_idx(c + 2, iq)
                    pltpu.async_copy(h_hbm.at[isrc[iq]], rows[rp], semg[rp])

            return 0

        lax.fori_loop(0, NCH // 4, quad, 0)
        plsc.subcore_barrier()
        pltpu.sync_copy(acc_sh.at[pl.ds(row0, rpt)],
                        out_hbm.at[cid, pl.ds(row0, rpt)])

    return prop_kernel


def _norm_scale(degp, x, R=1024):
    N, D = x.shape

    def body(degp_ref, x_ref, h_ref, norm_ref):
        deg = degp_ref[0] + degp_ref[1]
        norm = jnp.where(deg > 0, lax.rsqrt(jnp.maximum(deg, 1.0)), 0.0)
        h_ref[...] = x_ref[...] * norm
        norm_ref[...] = norm

    return pl.pallas_call(
        body,
        grid=(N // R,),
        in_specs=[
            pl.BlockSpec((NC, R, 1), lambda i: (0, i, 0)),
            pl.BlockSpec((R, D), lambda i: (i, 0)),
        ],
        out_specs=[
            pl.BlockSpec((R, D), lambda i: (i, 0)),
            pl.BlockSpec((R, 1), lambda i: (i, 0)),
        ],
        out_shape=[
            jax.ShapeDtypeStruct((N, D), jnp.float32),
            jax.ShapeDtypeStruct((N, 1), jnp.float32),
        ],
    )(degp, x)


def _combine_scale2(p, norm, R=1024):
    _, N, D = p.shape

    def body(p_ref, norm_ref, o_ref):
        n = norm_ref[...]
        o_ref[...] = (p_ref[0] + p_ref[1]) * (n * n)

    return pl.pallas_call(
        body,
        grid=(N // R,),
        in_specs=[
            pl.BlockSpec((NC, R, D), lambda i: (0, i, 0)),
            pl.BlockSpec((R, 1), lambda i: (i, 0)),
        ],
        out_specs=pl.BlockSpec((R, D), lambda i: (i, 0)),
        out_shape=jax.ShapeDtypeStruct((N, D), jnp.float32),
    )(p, norm)


def _combine_linear(p, norm, W, b, R=1024):
    _, N, D = p.shape
    DO = W.shape[1]

    def body(p_ref, norm_ref, w_ref, b_ref, o_ref):
        h = (p_ref[0] + p_ref[1]) * norm_ref[...]
        o_ref[...] = (
            jnp.dot(h, w_ref[...], preferred_element_type=jnp.float32)
            + b_ref[...]
        )

    return pl.pallas_call(
        body,
        grid=(N // R,),
        in_specs=[
            pl.BlockSpec((NC, R, D), lambda i: (0, i, 0)),
            pl.BlockSpec((R, 1), lambda i: (i, 0)),
            pl.BlockSpec((D, DO), lambda i: (0, 0)),
            pl.BlockSpec((1, DO), lambda i: (0, 0)),
        ],
        out_specs=pl.BlockSpec((R, DO), lambda i: (i, 0)),
        out_shape=jax.ShapeDtypeStruct((N, DO), jnp.float32),
    )(p, norm, W, b.reshape(1, DO))


def kernel(x, edge_index, W, b):
    N, D = x.shape
    E = edge_index.shape[1]
    C = 128
    NP = ((N + NS * 128 - 1) // (NS * 128)) * (NS * 128)
    PT = -(-E // (NW * 2 * C)) * 2 * C
    if NP == N and PT * NW != E:
        NP += NS * 128
    EP = PT * NW
    NCH = PT // C
    assert NCH % 2 == 0

    src = edge_index[0]
    dst = edge_index[1]
    npad = EP - E
    if npad:
        pad_src = (jnp.arange(npad, dtype=jnp.int32) % N)
        pad_dst = N + (jnp.arange(npad, dtype=jnp.int32) % (NP - N))
        src = jnp.concatenate([src, pad_src])
        dst = jnp.concatenate([dst, pad_dst])
    dst3 = dst.reshape(NW, NCH, C)
    xp = jnp.pad(x, ((0, NP - N), (0, 0))) if NP != N else x

    degp = _make_deg_kernel(NP, NCH, C)(dst3).reshape(NC, NP, 1)
    h1, norm = _norm_scale(degp, xp)
    prop = _make_prop_kernel(NP, D, NCH, C)
    p1 = prop(h1, src, dst)
    h2 = _combine_scale2(p1, norm)
    p2 = prop(h2, src, dst)
    out = _combine_linear(p2, norm, W, b)
    return out[:N] if NP != N else out
